# baseline (speedup 1.0000x reference)
import functools

import jax
import jax.numpy as jnp
from jax import lax
from jax.experimental import pallas as pl
from jax.experimental.pallas import tpu as pltpu
from jax.experimental.pallas import tpu_sc as plsc

EMBED = 64
HALF = EMBED // 2

_NC = 2
_NS = 16
_NW = _NC * _NS
_CHUNK = 640


def _table_body(ts_ref, out_ref):
    t = ts_ref[:, :]
    col = lax.broadcasted_iota(jnp.int32, out_ref.shape, 1)
    is_sin = col < HALF
    k = jnp.where(is_sin, col, col - HALF).astype(jnp.float32)
    inv_freq = jnp.exp(k * (-2.0 * jnp.log(10000.0) / EMBED))
    phase = t * inv_freq
    out_ref[:, :] = jnp.where(is_sin, jnp.sin(phase), jnp.cos(phase))


def _build_table(time_step):
    g = time_step.shape[0]
    return pl.pallas_call(
        _table_body,
        out_shape=jax.ShapeDtypeStruct((g, EMBED), jnp.float32),
    )(time_step.reshape(g, 1))


_SCCH = 2 * _CHUNK


def _gather_body(n_chunks_total, edge_hbm, batch_hbm, table_hbm,
                 out_hbm, idx_v, g_v, rows_v, sems):
    wid = lax.axis_index("s") * _NC + lax.axis_index("c")
    n_mine = (n_chunks_total - wid + _NW - 1) // _NW

    def step(i, carry):
        k = i * _NW + wid
        base = k * _SCCH
        rowbase = k * _CHUNK
        pltpu.sync_copy(edge_hbm.at[0, pl.ds(base, _SCCH)], idx_v)
        pltpu.async_copy(batch_hbm.at[idx_v], g_v, sems[0]).wait()
        pltpu.async_copy(table_hbm.at[g_v], rows_v, sems[1]).wait()
        pltpu.sync_copy(rows_v.at[pl.ds(0, _CHUNK)],
                        out_hbm.at[pl.ds(rowbase, _CHUNK),
                                   pl.ds(0, EMBED)])
        pltpu.sync_copy(rows_v.at[pl.ds(_CHUNK, _CHUNK)],
                        out_hbm.at[pl.ds(rowbase, _CHUNK),
                                   pl.ds(EMBED, EMBED)])
        return carry

    lax.fori_loop(0, n_mine, step, 0)


def _sc_gather(edge_index, batch, table):
    e = edge_index.shape[1]
    n_chunks_total = e // _SCCH
    mesh = plsc.VectorSubcoreMesh(core_axis_name="c", subcore_axis_name="s")
    run = pl.kernel(
        functools.partial(_gather_body, n_chunks_total),
        out_type=jax.ShapeDtypeStruct((e // 2, 2 * EMBED), jnp.float32),
        mesh=mesh,
        scratch_types=[
            pltpu.VMEM((_SCCH,), jnp.int32),
            pltpu.VMEM((_SCCH,), jnp.int32),
            pltpu.VMEM((_SCCH, EMBED), jnp.float32),
            [pltpu.SemaphoreType.DMA for _ in range(2)],
        ],
        compiler_params=pltpu.CompilerParams(use_tc_tiling_on_sc=False),
    )
    return run(edge_index, batch, table)


_RUNS = 25


def _relayout_body(in_ref, out_ref):
    xt = in_ref[:, :].T
    for s in range(_RUNS):
        c = s * _CHUNK
        out_ref[:, 2 * c:2 * c + _CHUNK] = xt[:EMBED, c:c + _CHUNK]
        out_ref[:, 2 * c + _CHUNK:2 * c + 2 * _CHUNK] = xt[EMBED:, c:c + _CHUNK]


def _tc_relayout(packed, e):
    rows = packed.shape[0]
    grid = (rows // (_RUNS * _CHUNK),)
    return pl.pallas_call(
        _relayout_body,
        grid=grid,
        in_specs=[pl.BlockSpec((_RUNS * _CHUNK, 2 * EMBED),
                               lambda i: (i, 0))],
        out_specs=pl.BlockSpec((EMBED, 2 * _RUNS * _CHUNK),
                               lambda i: (0, i)),
        out_shape=jax.ShapeDtypeStruct((EMBED, e), jnp.float32),
    )(packed)


def kernel(time_step, batch, edge_index):
    table = _build_table(time_step)
    e = edge_index.shape[1]
    pad = (-e) % (2 * _RUNS * _CHUNK)
    if pad:
        edge_index = jnp.concatenate(
            [edge_index,
             jnp.zeros((2, pad), dtype=edge_index.dtype)], axis=1)
    ep = e + pad
    packed = _sc_gather(edge_index, batch, table)
    out = _tc_relayout(packed, ep).T
    if pad:
        out = out[:e]
    return out


# speedup vs baseline: 1.0151x; 1.0151x over previous
import functools

import jax
import jax.numpy as jnp
from jax import lax
from jax.experimental import pallas as pl
from jax.experimental.pallas import tpu as pltpu
from jax.experimental.pallas import tpu_sc as plsc

EMBED = 64
HALF = EMBED // 2

_NC = 2
_NS = 16
_NW = _NC * _NS
_CHUNK = 640


def _table_body(ts_ref, out_ref):
    t = ts_ref[:, :]
    col = lax.broadcasted_iota(jnp.int32, out_ref.shape, 1)
    is_sin = col < HALF
    k = jnp.where(is_sin, col, col - HALF).astype(jnp.float32)
    inv_freq = jnp.exp(k * (-2.0 * jnp.log(10000.0) / EMBED))
    phase = t * inv_freq
    out_ref[:, :] = jnp.where(is_sin, jnp.sin(phase), jnp.cos(phase))


def _build_table(time_step):
    g = time_step.shape[0]
    return pl.pallas_call(
        _table_body,
        out_shape=jax.ShapeDtypeStruct((g, EMBED), jnp.float32),
    )(time_step.reshape(g, 1))


_SCCH = 2 * _CHUNK


def _gather_body(n_chunks_total, edge_hbm, batch_hbm, table_hbm,
                 out_hbm, idx_v, g_v, rows_v, sems):
    wid = lax.axis_index("s") * _NC + lax.axis_index("c")
    n_mine = (n_chunks_total - wid + _NW - 1) // _NW

    def step(i, carry):
        k = i * _NW + wid
        base = k * _SCCH
        rowbase = k * _CHUNK
        pltpu.sync_copy(edge_hbm.at[0, pl.ds(base, _SCCH)], idx_v)
        pltpu.async_copy(batch_hbm.at[idx_v], g_v, sems[0]).wait()
        pltpu.async_copy(table_hbm.at[g_v], rows_v, sems[1]).wait()
        w1 = pltpu.async_copy(rows_v.at[pl.ds(0, _CHUNK)],
                              out_hbm.at[pl.ds(rowbase, _CHUNK),
                                         pl.ds(0, EMBED)], sems[2])
        w2 = pltpu.async_copy(rows_v.at[pl.ds(_CHUNK, _CHUNK)],
                              out_hbm.at[pl.ds(rowbase, _CHUNK),
                                         pl.ds(EMBED, EMBED)], sems[3])
        w1.wait()
        w2.wait()
        return carry

    lax.fori_loop(0, n_mine, step, 0)


def _sc_gather(edge_index, batch, table):
    e = edge_index.shape[1]
    n_chunks_total = e // _SCCH
    mesh = plsc.VectorSubcoreMesh(core_axis_name="c", subcore_axis_name="s")
    run = pl.kernel(
        functools.partial(_gather_body, n_chunks_total),
        out_type=jax.ShapeDtypeStruct((e // 2, 2 * EMBED), jnp.float32),
        mesh=mesh,
        scratch_types=[
            pltpu.VMEM((_SCCH,), jnp.int32),
            pltpu.VMEM((_SCCH,), jnp.int32),
            pltpu.VMEM((_SCCH, EMBED), jnp.float32),
            [pltpu.SemaphoreType.DMA for _ in range(4)],
        ],
        compiler_params=pltpu.CompilerParams(use_tc_tiling_on_sc=False),
    )
    return run(edge_index, batch, table)


_RUNS = 25


def _relayout_body(in_ref, out_ref):
    xt = in_ref[:, :].T
    for s in range(_RUNS):
        c = s * _CHUNK
        out_ref[:, 2 * c:2 * c + _CHUNK] = xt[:EMBED, c:c + _CHUNK]
        out_ref[:, 2 * c + _CHUNK:2 * c + 2 * _CHUNK] = xt[EMBED:, c:c + _CHUNK]


def _tc_relayout(packed, e):
    rows = packed.shape[0]
    grid = (rows // (_RUNS * _CHUNK),)
    return pl.pallas_call(
        _relayout_body,
        grid=grid,
        in_specs=[pl.BlockSpec((_RUNS * _CHUNK, 2 * EMBED),
                               lambda i: (i, 0))],
        out_specs=pl.BlockSpec((EMBED, 2 * _RUNS * _CHUNK),
                               lambda i: (0, i)),
        out_shape=jax.ShapeDtypeStruct((EMBED, e), jnp.float32),
    )(packed)


def kernel(time_step, batch, edge_index):
    table = _build_table(time_step)
    e = edge_index.shape[1]
    pad = (-e) % (2 * _RUNS * _CHUNK)
    if pad:
        edge_index = jnp.concatenate(
            [edge_index,
             jnp.zeros((2, pad), dtype=edge_index.dtype)], axis=1)
    ep = e + pad
    packed = _sc_gather(edge_index, batch, table)
    out = _tc_relayout(packed, ep).T
    if pad:
        out = out[:e]
    return out
